# baseline (speedup 1.0000x reference)
import functools
import jax
import jax.numpy as jnp
from jax import lax
from jax.experimental import pallas as pl
from jax.experimental.pallas import tpu as pltpu
from jax.experimental.pallas import tpu_sc as plsc

CH, IT, TR, ST = 64, 32, 32, 12
L = 50
NX = 23
OW = CH + 3 * IT + 7 * TR + ST
NROWS = 4096 * L
NW = 32
RPW = NROWS // NW
CR = 128
NCHUNK = RPW // CR
XW = CR * NX
OWW = CR * OW

_ITEM_BASE = CH
_TRAIT_BASE = CH + 3 * IT


def _sc_body(xf, comb, out, xv, ob, vc, offs):
    wid = lax.axis_index("s") * 2 + lax.axis_index("c")
    pltpu.sync_copy(comb, vc)
    champ = [vc[pl.ds(16 * k, 16)] for k in range(CH // 16)]
    ivec = lax.iota(jnp.int32, 16)
    is_item = (ivec >= 1) & (ivec <= 3)
    maxv = jnp.where(is_item, 2, 6)
    basev = jnp.where(is_item, _ITEM_BASE, _TRAIT_BASE)

    def chunk_body(c, _):
        base_row = wid * RPW + c * CR
        pltpu.sync_copy(xf.at[pl.ds(base_row * NX, XW)], xv.at[pl.ds(0, XW)])

        def row_body(r, _):
            ob_off = r * OW
            for k in range(CH // 16):
                ob[pl.ds(ob_off + 16 * k, 16)] = champ[k]
            idvec = xv[pl.ds(r * NX, 16)]
            offv = basev + jnp.clip(idvec.astype(jnp.int32), 0, maxv) * 32
            for s in range(10):
                src = offv[1 + s]
                dst = ob_off + CH + 32 * s
                ob[pl.ds(dst, 16)] = vc[pl.ds(src, 16)]
                ob[pl.ds(dst + 16, 16)] = vc[pl.ds(src + 16, 16)]
            ob[pl.ds(ob_off + CH + 320, 16)] = xv[pl.ds(r * NX + 11, 16)]
            return _

        lax.fori_loop(0, CR, row_body, 0, unroll=False)
        pltpu.sync_copy(ob.at[pl.ds(0, OWW)],
                        out.at[pl.ds(base_row * OW, OWW)])
        return _

    lax.fori_loop(0, NCHUNK, chunk_body, 0, unroll=False)


def kernel(x, champion_w, item_w, trait_w):
    xf = x.reshape(-1)
    comb = jnp.concatenate([
        champion_w.reshape(-1), item_w.reshape(-1), trait_w.reshape(-1)])
    mesh = plsc.VectorSubcoreMesh(core_axis_name="c", subcore_axis_name="s")
    out_flat = pl.kernel(
        _sc_body,
        mesh=mesh,
        out_type=jax.ShapeDtypeStruct((NROWS * OW,), jnp.float32),
        scratch_types=[
            pltpu.VMEM((XW + 16,), jnp.float32),
            pltpu.VMEM((OWW + 16,), jnp.float32),
            pltpu.VMEM((384,), jnp.float32),
            pltpu.VMEM((16 * CR,), jnp.int32),
        ],
    )(xf, comb)
    return out_flat.reshape(4096, L, OW)


# speedup vs baseline: 1.0089x; 1.0089x over previous
import functools
import jax
import jax.numpy as jnp
from jax import lax
from jax.experimental import pallas as pl
from jax.experimental.pallas import tpu as pltpu
from jax.experimental.pallas import tpu_sc as plsc

CH, IT, TR, ST = 64, 32, 32, 12
L = 50
NX = 23
OW = CH + 3 * IT + 7 * TR + ST
NROWS = 4096 * L
NW = 32
RPW = NROWS // NW
CR = 128
NCHUNK = RPW // CR
XW = CR * NX
OWW = CR * OW

_ITEM_BASE = CH
_TRAIT_BASE = CH + 3 * IT


def _sc_body(xf, comb, out, xv, ob, vc, offs):
    wid = lax.axis_index("s") * 2 + lax.axis_index("c")
    pltpu.sync_copy(comb, vc)
    champ = [vc[pl.ds(16 * k, 16)] for k in range(CH // 16)]
    ivec = lax.iota(jnp.int32, 16)
    is_item = (ivec >= 1) & (ivec <= 3)
    maxv = jnp.where(is_item, 2, 6)
    basev = jnp.where(is_item, _ITEM_BASE, _TRAIT_BASE)
    def champ_body(r, _):
        for k in range(1, CH // 16):
            ob[pl.ds(r * OW + 16 * k, 16)] = champ[k]
        return _

    lax.fori_loop(0, CR, champ_body, 0, unroll=4)

    def chunk_body(c, _):
        base_row = wid * RPW + c * CR
        pltpu.sync_copy(xf.at[pl.ds(base_row * NX, XW)], xv.at[pl.ds(0, XW)])

        def row_body(r, _):
            ob_off = r * OW
            ob[pl.ds(ob_off, 16)] = champ[0]
            idvec = xv[pl.ds(r * NX, 16)]
            offv = basev + jnp.clip(idvec.astype(jnp.int32), 0, maxv) * 32
            for s in range(10):
                src = offv[1 + s]
                dst = ob_off + CH + 32 * s
                ob[pl.ds(dst, 16)] = vc[pl.ds(src, 16)]
                ob[pl.ds(dst + 16, 16)] = vc[pl.ds(src + 16, 16)]
            ob[pl.ds(ob_off + CH + 320, 16)] = xv[pl.ds(r * NX + 11, 16)]
            return _

        lax.fori_loop(0, CR, row_body, 0, unroll=4)
        pltpu.sync_copy(ob.at[pl.ds(0, OWW)],
                        out.at[pl.ds(base_row * OW, OWW)])
        return _

    lax.fori_loop(0, NCHUNK, chunk_body, 0, unroll=False)


def kernel(x, champion_w, item_w, trait_w):
    xf = x.reshape(-1)
    comb = jnp.concatenate([
        champion_w.reshape(-1), item_w.reshape(-1), trait_w.reshape(-1)])
    mesh = plsc.VectorSubcoreMesh(core_axis_name="c", subcore_axis_name="s")
    out_flat = pl.kernel(
        _sc_body,
        mesh=mesh,
        out_type=jax.ShapeDtypeStruct((NROWS * OW,), jnp.float32),
        scratch_types=[
            pltpu.VMEM((XW + 16,), jnp.float32),
            pltpu.VMEM((OWW + 16,), jnp.float32),
            pltpu.VMEM((384,), jnp.float32),
            pltpu.VMEM((16 * CR,), jnp.int32),
        ],
    )(xf, comb)
    return out_flat.reshape(4096, L, OW)


# speedup vs baseline: 2.6345x; 2.6113x over previous
import numpy as np
import jax
import jax.numpy as jnp
from jax.experimental import pallas as pl
from jax.experimental.pallas import tpu as pltpu

CH, IT, TR, ST = 64, 32, 32, 12
L = 50
NID = 11
NX = NID + ST
OW = CH + 3 * IT + 7 * TR + ST
C = 128

_SLOT_ROWS = [3, 3, 3, 7, 7, 7, 7, 7, 7, 7]
_SLOT_OFF = [CH + 32 * i for i in range(10)]
_BIG = np.float32(1e30)


def _consts():
    E = np.zeros((NX, C), np.float32)
    K = np.full((C,), _BIG, np.float32)
    K2 = np.full((C,), _BIG, np.float32)
    K[0], K2[0] = -_BIG, _BIG
    for s in range(10):
        nr = _SLOT_ROWS[s]
        for k in range(8):
            j = 1 + s * 8 + k
            if k < nr:
                E[1 + s, j] = 1.0
                K[j] = -_BIG if k == 0 else np.float32(k)
                K2[j] = _BIG if k == nr - 1 else np.float32(k + 1)
    return jnp.asarray(E), jnp.asarray(K), jnp.asarray(K2)


def _mixmat(champion_w, item_w, trait_w):
    M = jnp.zeros((C, OW), jnp.float32)
    M = M.at[0, :CH].set(champion_w[0])
    for s in range(10):
        tab = item_w if s < 3 else trait_w
        nr = _SLOT_ROWS[s]
        off = _SLOT_OFF[s]
        M = M.at[1 + s * 8:1 + s * 8 + nr, off:off + 32].set(tab)
    return M


def _body(x_ref, e_ref, k_ref, k2_ref, m_ref, o_ref):
    x = x_ref[...]
    idsf = jnp.clip(jnp.floor(x), 0.0, 7.0)
    spread = jax.lax.dot_general(
        idsf, e_ref[...],
        dimension_numbers=(((2,), (0,)), ((), ())),
        preferred_element_type=jnp.float32,
    )
    k = k_ref[...].reshape(1, 1, C)
    k2 = k2_ref[...].reshape(1, 1, C)
    hot = jnp.where((spread >= k) & (spread < k2), 1.0, 0.0)
    emb = jax.lax.dot_general(
        hot, m_ref[...],
        dimension_numbers=(((2,), (0,)), ((), ())),
        preferred_element_type=jnp.float32,
    )
    o_ref[...] = emb
    o_ref[:, :, CH + 320:] = x[:, :, NID:]


def kernel(x, champion_w, item_w, trait_w):
    B = x.shape[0]
    rb = 64
    E, K, K2 = _consts()
    M = _mixmat(champion_w, item_w, trait_w)
    return pl.pallas_call(
        _body,
        grid=(B // rb,),
        in_specs=[
            pl.BlockSpec((rb, L, NX), lambda i: (i, 0, 0)),
            pl.BlockSpec((NX, C), lambda i: (0, 0)),
            pl.BlockSpec((C,), lambda i: (0,)),
            pl.BlockSpec((C,), lambda i: (0,)),
            pl.BlockSpec((C, OW), lambda i: (0, 0)),
        ],
        out_specs=pl.BlockSpec((rb, L, OW), lambda i: (i, 0, 0)),
        out_shape=jax.ShapeDtypeStruct((B, L, OW), x.dtype),
        compiler_params=pltpu.CompilerParams(
            dimension_semantics=("arbitrary",),
        ),
    )(x, E, K, K2, M)
